# baseline (speedup 1.0000x reference)
import functools

import jax
import jax.numpy as jnp
from jax import lax
from jax.experimental import pallas as pl
from jax.experimental.pallas import tpu as pltpu
from jax.experimental.pallas import tpu_sc as plsc

D_MODEL = 64
D_PAD = 128
NH = 2
DH = D_MODEL // NH

NC = 2
NS = 16
NW = NC * NS

GS = (96, 104)
RW = 2
NB = 2


@jax.jit
def _gather_rows(w0, w1, x):
  n_rows, seq = x.shape
  r_per_w = n_rows // NW
  n_outer = r_per_w // (RW * NB)
  halves = (w0, w1)

  mesh = plsc.VectorSubcoreMesh(
      core_axis_name="c", subcore_axis_name="s", num_cores=NC, num_subcores=NS
  )

  @functools.partial(
      pl.kernel,
      mesh=mesh,
      compiler_params=pltpu.CompilerParams(use_tc_tiling_on_sc=False),
      out_type=jax.ShapeDtypeStruct((n_rows * seq, D_PAD), jnp.float32),
      scratch_types=(
          [pltpu.VMEM((r_per_w, seq), jnp.int32)]
          + [pltpu.VMEM((NB, RW * seq, DH), jnp.float32)] * NH
          + [pltpu.SemaphoreType.DMA] * (NH * (NB * RW * 2 + NB))
      ),
  )
  def k(t0_hbm, t1_hbm, idx_hbm, out_hbm, idx_v, *rest):
    rows_v = rest[:NH]
    sems = rest[NH:]
    tables = (t0_hbm, t1_hbm)
    gsem = [sems[hh * (NB * RW * 2):(hh + 1) * (NB * RW * 2)]
            for hh in range(NH)]
    osem = sems[NH * NB * RW * 2:]
    wid = lax.axis_index("s") * NC + lax.axis_index("c")
    base = wid * r_per_w
    pltpu.sync_copy(idx_hbm.at[pl.ds(base, r_per_w)], idx_v)

    def outer(i, carry):
      r0 = i * NB * RW
      for o in range(NB):
        @pl.when(i > 0)
        def _wait_out():
          for hh in range(NH):
            pltpu.make_async_copy(
                rows_v[hh].at[o],
                out_hbm.at[pl.ds((base + r0 + o * RW - NB * RW) * seq,
                                 RW * seq),
                           pl.ds(hh * DH, DH)],
                osem[hh * NB + o],
            ).wait()

        for j in range(RW):
          r = r0 + o * RW + j
          for h, (off, g) in enumerate(zip((0, GS[0]), GS)):
            for hh in range(NH):
              pltpu.async_copy(
                  tables[hh].at[idx_v.at[r, pl.ds(off, g)]],
                  rows_v[hh].at[o, pl.ds(j * seq + off, g)],
                  gsem[hh][(o * RW + j) * 2 + h])
      for o in range(NB):
        for j in range(RW):
          r = r0 + o * RW + j
          for h, (off, g) in enumerate(zip((0, GS[0]), GS)):
            for hh in range(NH):
              pltpu.make_async_copy(
                  tables[hh].at[idx_v.at[r, pl.ds(off, g)]],
                  rows_v[hh].at[o, pl.ds(j * seq + off, g)],
                  gsem[hh][(o * RW + j) * 2 + h]).wait()
        for hh in range(NH):
          pltpu.async_copy(
              rows_v[hh].at[o],
              out_hbm.at[pl.ds((base + r0 + o * RW) * seq, RW * seq),
                         pl.ds(hh * DH, DH)],
              osem[hh * NB + o])
      return carry

    lax.fori_loop(0, n_outer, outer, 0)
    for o in range(NB):
      for hh in range(NH):
        pltpu.make_async_copy(
            rows_v[hh].at[o],
            out_hbm.at[pl.ds((base + (n_outer - 1) * NB * RW + o * RW) * seq,
                             RW * seq),
                       pl.ds(hh * DH, DH)],
            osem[hh * NB + o],
        ).wait()

  return k(w0, w1, x)


def kernel(x, weight):
  n_rows, seq = x.shape
  xi = x.astype(jnp.int32)
  out128 = _gather_rows(weight[:, :DH], weight[:, DH:], xi)
  return out128[:, :D_MODEL].reshape(n_rows, seq, D_MODEL)
